# Initial kernel scaffold; baseline (speedup 1.0000x reference)
import jax
import jax.numpy as jnp
from jax.experimental import pallas as pl


def kernel(x, edge_index, W_kqv, b_kqv, W_att1, b_att1, W_att2, b_att2, W_out, b_out):
    raise NotImplementedError("write your pallas kernel here")


# speedup vs baseline: 5.2788x; 5.2788x over previous
import functools
import math

import jax
import jax.numpy as jnp
from jax import lax
from jax.experimental import pallas as pl
from jax.experimental.pallas import tpu as pltpu
from jax.experimental.pallas import tpu_sc as plsc

N = 10000
EMB = 128
HD = 64
N_PAD = 10240
NC, NS = 2, 16
C = 128
K = 318
E_SUB = C * K
E_PAD = E_SUB * NS
N_ACC = 10112
ROWS_PER_SUB = N_ACC // NS


def _proj_kernel(x_ref, wkqv_ref, bkqv_ref, watt1_ref, batt1_ref,
                 s_ref, aq_ref):
    wk = watt1_ref[0:HD, :]
    wq = watt1_ref[HD:2 * HD, :]
    w = wkqv_ref[...]
    b = bkqv_ref[...]
    scale = 1.0 / math.sqrt(HD)
    dot = functools.partial(jnp.dot, preferred_element_type=jnp.float32)
    wcat = jnp.concatenate([
        dot(w[:, 128:192] * scale, wk),
        w[:, 256:320],
        dot(w[:, 192:256] * scale, wk),
        w[:, 320:384],
        dot(w[:, 0:64], wq),
        dot(w[:, 64:128], wq),
    ], axis=1)
    b1 = batt1_ref[...]
    bcat = jnp.concatenate([
        dot(b[:, 128:192] * scale, wk),
        b[:, 256:320],
        dot(b[:, 192:256] * scale, wk),
        b[:, 320:384],
        dot(b[:, 0:64], wq) + b1,
        dot(b[:, 64:128], wq) + b1,
    ], axis=1)
    t = dot(x_ref[...], wcat) + bcat
    s_ref[0] = t[:, 0:128]
    s_ref[1] = t[:, 128:256]
    aq_ref[...] = t[:, 256:384]


def _project(x_pad, w_kqv, b_kqv, w_att1, b_att1):
    blk = 512
    grid = (N_PAD // blk,)
    full = lambda shape: pl.BlockSpec(shape, lambda i: tuple(0 for _ in shape))
    return pl.pallas_call(
        _proj_kernel,
        grid=grid,
        in_specs=[
            pl.BlockSpec((blk, EMB), lambda i: (i, 0)),
            full((EMB, 3 * EMB)),
            full((1, 3 * EMB)),
            full((2 * HD, HD)),
            full((1, HD)),
        ],
        out_specs=[
            pl.BlockSpec((NC, blk, EMB), lambda i: (0, i, 0)),
            pl.BlockSpec((blk, EMB), lambda i: (i, 0)),
        ],
        out_shape=[
            jax.ShapeDtypeStruct((NC, N_PAD, EMB), jnp.float32),
            jax.ShapeDtypeStruct((N_PAD, EMB), jnp.float32),
        ],
    )(x_pad, w_kqv, b_kqv, w_att1, b_att1)


def _edge_kernel(s_hbm, aq_hbm, src_hbm, dst_hbm, w2_hbm, z_hbm,
                 acc_hbm,
                 src_v, dst_v, s_rows, aq_rows, stage, w2_v,
                 acc_sh, sem_s, sem_q):
    cc = lax.axis_index("c")
    sid = lax.axis_index("s")
    sub_base = sid * E_SUB

    pltpu.sync_copy(z_hbm, acc_sh.at[pl.ds(sid * ROWS_PER_SUB, ROWS_PER_SUB)])
    pltpu.sync_copy(w2_hbm, w2_v)
    plsc.subcore_barrier()

    w2v = [w2_v[pl.ds(16 * j, 16)] for j in range(4)]
    lane = lax.iota(jnp.int32, 16)
    perms = [jnp.bitwise_xor(lane, sh) for sh in (8, 4, 2, 1)]
    aq_off = cc * HD

    def zero_body(e, carry):
        for j in range(5, 8):
            stage[e, pl.ds(16 * j, 16)] = jnp.zeros((16,), jnp.float32)
        return carry
    lax.fori_loop(0, C, zero_body, 0)

    def hsum(u):
        for p in perms:
            u = u + u.at[p].get(mode="promise_in_bounds")
        return u

    def edge_body(e, carry):
        t = [jnp.maximum(s_rows[e, pl.ds(16 * j, 16)]
                         + aq_rows[e, pl.ds(aq_off + 16 * j, 16)], 0.0)
             * w2v[j] for j in range(4)]
        wv = jnp.exp(hsum((t[0] + t[1]) + (t[2] + t[3])))
        for j in range(4):
            stage[e, pl.ds(16 * j, 16)] = s_rows[e, pl.ds(64 + 16 * j, 16)] * wv
        tail = jnp.where(lane == 0, wv,
                         jnp.where(lane == 1, 1.0, 0.0))
        stage[e, pl.ds(64, 16)] = tail
        return carry

    def chunk_body(k, carry):
        base = sub_base + k * C
        pltpu.sync_copy(src_hbm.at[pl.ds(base, C)], src_v)
        pltpu.sync_copy(dst_hbm.at[pl.ds(base, C)], dst_v)
        cp1 = pltpu.async_copy(s_hbm.at[cc].at[src_v], s_rows, sem_s)
        cp2 = pltpu.async_copy(aq_hbm.at[dst_v], aq_rows, sem_q)
        cp1.wait()
        cp2.wait()
        lax.fori_loop(0, C, edge_body, 0)
        pltpu.sync_copy(stage, acc_sh.at[dst_v], add=True)
        return carry

    lax.fori_loop(0, K, chunk_body, 0)
    plsc.subcore_barrier()
    pltpu.sync_copy(acc_sh.at[pl.ds(sid * ROWS_PER_SUB, ROWS_PER_SUB)],
                    acc_hbm.at[cc, pl.ds(sid * ROWS_PER_SUB, ROWS_PER_SUB)])


def _edge_pass(s_tab, aq_tab, src, dst, w2t, zrows):
    mesh = plsc.VectorSubcoreMesh(core_axis_name="c", subcore_axis_name="s")
    f = pl.kernel(
        _edge_kernel,
        out_type=jax.ShapeDtypeStruct((NC, N_PAD, EMB), jnp.float32),
        mesh=mesh,
        scratch_types=[
            pltpu.VMEM((C,), jnp.int32),
            pltpu.VMEM((C,), jnp.int32),
            pltpu.VMEM((C, EMB), jnp.float32),
            pltpu.VMEM((C, EMB), jnp.float32),
            pltpu.VMEM((C, EMB), jnp.float32),
            pltpu.VMEM((HD,), jnp.float32),
            pltpu.VMEM_SHARED((N_ACC, EMB), jnp.float32),
            pltpu.SemaphoreType.DMA,
            pltpu.SemaphoreType.DMA,
        ],
    )
    return f(s_tab, aq_tab, src, dst, w2t, zrows)


def _out_kernel(acc_ref, x_ref, wout_ref, bout_ref, o_ref):
    a0 = acc_ref[0]
    a1 = acc_ref[1]
    norm = jnp.concatenate([a0[:, 0:64] / (a0[:, 64:65] + 1e-16),
                            a1[:, 0:64] / (a1[:, 64:65] + 1e-16)], axis=1)
    cnt = a0[:, 65:66]
    o = (jnp.dot(norm, wout_ref[...], preferred_element_type=jnp.float32)
         + cnt * bout_ref[...])
    o_ref[...] = jnp.maximum(o, 0.0) + x_ref[...]


def _finish(acc, x_pad, w_out, b_out):
    blk = 512
    grid = (N_PAD // blk,)
    return pl.pallas_call(
        _out_kernel,
        grid=grid,
        in_specs=[
            pl.BlockSpec((NC, blk, EMB), lambda i: (0, i, 0)),
            pl.BlockSpec((blk, EMB), lambda i: (i, 0)),
            pl.BlockSpec((EMB, EMB), lambda i: (0, 0)),
            pl.BlockSpec((1, EMB), lambda i: (0, 0)),
        ],
        out_specs=pl.BlockSpec((blk, EMB), lambda i: (i, 0)),
        out_shape=jax.ShapeDtypeStruct((N_PAD, EMB), jnp.float32),
    )(acc, x_pad, w_out, b_out)


def kernel(x, edge_index, W_kqv, b_kqv, W_att1, b_att1, W_att2, b_att2,
           W_out, b_out):
    n = x.shape[0]
    ei = jnp.concatenate([edge_index, edge_index[::-1]], axis=1)
    mask = ei[0] != ei[1]
    src = jnp.where(mask, ei[0], jnp.zeros((), jnp.int32))
    dst = jnp.where(mask, ei[1], jnp.asarray(n, jnp.int32))
    loops = jnp.arange(n, dtype=jnp.int32)
    src = jnp.concatenate([src, loops])
    dst = jnp.concatenate([dst, loops])
    pad = E_PAD - src.shape[0]
    src = jnp.concatenate([src, jnp.zeros((pad,), jnp.int32)])
    dst = jnp.concatenate([dst, jnp.full((pad,), n + 1, jnp.int32)])

    x_pad = jnp.pad(x, ((0, N_PAD - n), (0, 0)))
    w2t = W_att2[:, 0]
    zrows = jnp.zeros((ROWS_PER_SUB, EMB), jnp.float32)

    s_tab, aq_tab = _project(x_pad, W_kqv, b_kqv.reshape(1, -1),
                             W_att1, b_att1.reshape(1, -1))
    acc = _edge_pass(s_tab, aq_tab, src, dst, w2t, zrows)
    out = _finish(acc, x_pad, W_out, b_out.reshape(1, -1))
    return out[:n]


# speedup vs baseline: 8.2879x; 1.5700x over previous
import functools
import math

import jax
import jax.numpy as jnp
from jax import lax
from jax.experimental import pallas as pl
from jax.experimental.pallas import tpu as pltpu
from jax.experimental.pallas import tpu_sc as plsc

N = 10000
EMB = 128
HD = 64
N_PAD = 10240
NC, NS = 2, 16
C = 72
K = 566
E_SUB = C * K
E_PAD = E_SUB * NS
N_ACC = 10112
ROWS_PER_SUB = N_ACC // NS


def _proj_kernel(x_ref, wkqv_ref, bkqv_ref, watt1_ref, batt1_ref,
                 s_ref, aq_ref):
    wk = watt1_ref[0:HD, :]
    wq = watt1_ref[HD:2 * HD, :]
    w = wkqv_ref[...]
    b = bkqv_ref[...]
    scale = 1.0 / math.sqrt(HD)
    dot = functools.partial(jnp.dot, preferred_element_type=jnp.float32)
    wcat = jnp.concatenate([
        dot(w[:, 128:192] * scale, wk),
        w[:, 256:320],
        dot(w[:, 192:256] * scale, wk),
        w[:, 320:384],
        dot(w[:, 0:64], wq),
        dot(w[:, 64:128], wq),
    ], axis=1)
    b1 = batt1_ref[...]
    bcat = jnp.concatenate([
        dot(b[:, 128:192] * scale, wk),
        b[:, 256:320],
        dot(b[:, 192:256] * scale, wk),
        b[:, 320:384],
        dot(b[:, 0:64], wq) + b1,
        dot(b[:, 64:128], wq) + b1,
    ], axis=1)
    t = dot(x_ref[...], wcat) + bcat
    s_ref[0] = t[:, 0:128]
    s_ref[1] = t[:, 128:256]
    aq_ref[...] = t[:, 256:384]


def _project(x_pad, w_kqv, b_kqv, w_att1, b_att1):
    blk = 512
    grid = (N_PAD // blk,)
    full = lambda shape: pl.BlockSpec(shape, lambda i: tuple(0 for _ in shape))
    return pl.pallas_call(
        _proj_kernel,
        grid=grid,
        in_specs=[
            pl.BlockSpec((blk, EMB), lambda i: (i, 0)),
            full((EMB, 3 * EMB)),
            full((1, 3 * EMB)),
            full((2 * HD, HD)),
            full((1, HD)),
        ],
        out_specs=[
            pl.BlockSpec((NC, blk, EMB), lambda i: (0, i, 0)),
            pl.BlockSpec((blk, EMB), lambda i: (i, 0)),
        ],
        out_shape=[
            jax.ShapeDtypeStruct((NC, N_PAD, EMB), jnp.float32),
            jax.ShapeDtypeStruct((N_PAD, EMB), jnp.float32),
        ],
    )(x_pad, w_kqv, b_kqv, w_att1, b_att1)


def _edge_kernel(s_hbm, aq_hbm, src_hbm, dst_hbm, w2_hbm, z_hbm,
                 acc_hbm,
                 src_v, dst_v, s_rows, aq_rows, stage, w2_v,
                 acc_sh, sem_s, sem_q):
    cc = lax.axis_index("c")
    sid = lax.axis_index("s")
    sub_base = sid * E_SUB

    pltpu.sync_copy(z_hbm, acc_sh.at[pl.ds(sid * ROWS_PER_SUB, ROWS_PER_SUB)])
    pltpu.sync_copy(w2_hbm, w2_v)
    plsc.subcore_barrier()

    w2v = [w2_v[pl.ds(16 * j, 16)] for j in range(4)]
    lane = lax.iota(jnp.int32, 16)
    perms = [jnp.bitwise_xor(lane, sh) for sh in (8, 4, 2, 1)]
    tail_base = jnp.where(lane == 1, 1.0, 0.0).astype(jnp.float32)
    aq_off = cc * HD

    @plsc.parallel_loop(0, C)
    def zero_body(e):
        for j in range(5, 8):
            stage[e, pl.ds(16 * j, 16)] = jnp.zeros((16,), jnp.float32)

    def hsum(u):
        for p in perms:
            u = u + u.at[p].get(mode="promise_in_bounds")
        return u

    def fetch(k, b):
        base = sub_base + k * C
        pltpu.sync_copy(src_hbm.at[pl.ds(base, C)], src_v.at[b])
        pltpu.sync_copy(dst_hbm.at[pl.ds(base, C)], dst_v.at[b])
        pltpu.async_copy(s_hbm.at[cc].at[src_v.at[b]], s_rows.at[b], sem_s)
        pltpu.async_copy(aq_hbm.at[dst_v.at[b]], aq_rows.at[b], sem_q)

    def wait_gather(b):
        pltpu.make_async_copy(s_hbm.at[cc].at[src_v.at[b]],
                              s_rows.at[b], sem_s).wait()
        pltpu.make_async_copy(aq_hbm.at[dst_v.at[b]],
                              aq_rows.at[b], sem_q).wait()

    def compute(b):
        @plsc.parallel_loop(0, C, unroll=4)
        def edge_body(e):
            t = [jnp.maximum(s_rows[b, e, pl.ds(16 * j, 16)]
                             + aq_rows[b, e, pl.ds(aq_off + 16 * j, 16)], 0.0)
                 * w2v[j] for j in range(4)]
            wv = jnp.exp(hsum((t[0] + t[1]) + (t[2] + t[3])))
            for j in range(4):
                stage[e, pl.ds(16 * j, 16)] = (
                    s_rows[b, e, pl.ds(64 + 16 * j, 16)] * wv)
            stage[e, pl.ds(64, 16)] = jnp.where(lane == 0, wv, tail_base)

    fetch(0, 0)
    fetch(1, 1)

    def pair_body(p, carry):
        k0 = p * 2
        for b in range(2):
            k = k0 + b
            wait_gather(b)
            compute(b)
            pltpu.sync_copy(stage, acc_sh.at[dst_v.at[b]], add=True)

            @pl.when(k + 2 < K)
            def _():
                fetch(k + 2, b)
        return carry

    lax.fori_loop(0, K // 2, pair_body, 0)
    plsc.subcore_barrier()
    pltpu.sync_copy(acc_sh.at[pl.ds(sid * ROWS_PER_SUB, ROWS_PER_SUB)],
                    acc_hbm.at[cc, pl.ds(sid * ROWS_PER_SUB, ROWS_PER_SUB)])


def _edge_pass(s_tab, aq_tab, src, dst, w2t, zrows):
    mesh = plsc.VectorSubcoreMesh(core_axis_name="c", subcore_axis_name="s")
    f = pl.kernel(
        _edge_kernel,
        out_type=jax.ShapeDtypeStruct((NC, N_PAD, EMB), jnp.float32),
        mesh=mesh,
        scratch_types=[
            pltpu.VMEM((2, C), jnp.int32),
            pltpu.VMEM((2, C), jnp.int32),
            pltpu.VMEM((2, C, EMB), jnp.float32),
            pltpu.VMEM((2, C, EMB), jnp.float32),
            pltpu.VMEM((C, EMB), jnp.float32),
            pltpu.VMEM((HD,), jnp.float32),
            pltpu.VMEM_SHARED((N_ACC, EMB), jnp.float32),
            pltpu.SemaphoreType.DMA,
            pltpu.SemaphoreType.DMA,
        ],
    )
    return f(s_tab, aq_tab, src, dst, w2t, zrows)


def _out_kernel(acc_ref, x_ref, wout_ref, bout_ref, o_ref):
    a0 = acc_ref[0]
    a1 = acc_ref[1]
    norm = jnp.concatenate([a0[:, 0:64] / (a0[:, 64:65] + 1e-16),
                            a1[:, 0:64] / (a1[:, 64:65] + 1e-16)], axis=1)
    cnt = a0[:, 65:66]
    o = (jnp.dot(norm, wout_ref[...], preferred_element_type=jnp.float32)
         + cnt * bout_ref[...])
    o_ref[...] = jnp.maximum(o, 0.0) + x_ref[...]


def _finish(acc, x_pad, w_out, b_out):
    blk = 512
    grid = (N_PAD // blk,)
    return pl.pallas_call(
        _out_kernel,
        grid=grid,
        in_specs=[
            pl.BlockSpec((NC, blk, EMB), lambda i: (0, i, 0)),
            pl.BlockSpec((blk, EMB), lambda i: (i, 0)),
            pl.BlockSpec((EMB, EMB), lambda i: (0, 0)),
            pl.BlockSpec((1, EMB), lambda i: (0, 0)),
        ],
        out_specs=pl.BlockSpec((blk, EMB), lambda i: (i, 0)),
        out_shape=jax.ShapeDtypeStruct((N_PAD, EMB), jnp.float32),
    )(acc, x_pad, w_out, b_out)


def kernel(x, edge_index, W_kqv, b_kqv, W_att1, b_att1, W_att2, b_att2,
           W_out, b_out):
    n = x.shape[0]
    ei = jnp.concatenate([edge_index, edge_index[::-1]], axis=1)
    mask = ei[0] != ei[1]
    src = jnp.where(mask, ei[0], jnp.zeros((), jnp.int32))
    dst = jnp.where(mask, ei[1], jnp.asarray(n, jnp.int32))
    loops = jnp.arange(n, dtype=jnp.int32)
    src = jnp.concatenate([src, loops])
    dst = jnp.concatenate([dst, loops])
    pad = E_PAD - src.shape[0]
    src = jnp.concatenate([src, jnp.zeros((pad,), jnp.int32)])
    dst = jnp.concatenate([dst, jnp.full((pad,), n + 1, jnp.int32)])

    x_pad = jnp.pad(x, ((0, N_PAD - n), (0, 0)))
    w2t = W_att2[:, 0]
    zrows = jnp.zeros((ROWS_PER_SUB, EMB), jnp.float32)

    s_tab, aq_tab = _project(x_pad, W_kqv, b_kqv.reshape(1, -1),
                             W_att1, b_att1.reshape(1, -1))
    acc = _edge_pass(s_tab, aq_tab, src, dst, w2t, zrows)
    out = _finish(acc, x_pad, W_out, b_out.reshape(1, -1))
    return out[:n]


# speedup vs baseline: 8.6338x; 1.0417x over previous
import functools
import math

import jax
import jax.numpy as jnp
from jax import lax
from jax.experimental import pallas as pl
from jax.experimental.pallas import tpu as pltpu
from jax.experimental.pallas import tpu_sc as plsc

N = 10000
EMB = 128
HD = 64
N_PAD = 10240
NC, NS = 2, 16
C = 80
K = 512
E_SUB = C * K
E_PAD = E_SUB * NS
N_ACC = 10112
ROWS_PER_SUB = N_ACC // NS


def _proj_kernel(x_ref, wkqv_ref, bkqv_ref, watt1_ref, batt1_ref,
                 s_ref, aq_ref):
    wk = watt1_ref[0:HD, :]
    wq = watt1_ref[HD:2 * HD, :]
    w = wkqv_ref[...]
    b = bkqv_ref[...]
    scale = 1.0 / math.sqrt(HD)
    dot = functools.partial(jnp.dot, preferred_element_type=jnp.float32)
    wcat = jnp.concatenate([
        dot(w[:, 128:192] * scale, wk),
        w[:, 256:320],
        dot(w[:, 192:256] * scale, wk),
        w[:, 320:384],
        dot(w[:, 0:64], wq),
        dot(w[:, 64:128], wq),
    ], axis=1)
    b1 = batt1_ref[...]
    bcat = jnp.concatenate([
        dot(b[:, 128:192] * scale, wk),
        b[:, 256:320],
        dot(b[:, 192:256] * scale, wk),
        b[:, 320:384],
        dot(b[:, 0:64], wq) + b1,
        dot(b[:, 64:128], wq) + b1,
    ], axis=1)
    t = dot(x_ref[...], wcat) + bcat
    s_ref[0] = t[:, 0:128]
    s_ref[1] = t[:, 128:256]
    aq_ref[...] = t[:, 256:384]


def _project(x_pad, w_kqv, b_kqv, w_att1, b_att1):
    blk = 512
    grid = (N_PAD // blk,)
    full = lambda shape: pl.BlockSpec(shape, lambda i: tuple(0 for _ in shape))
    return pl.pallas_call(
        _proj_kernel,
        grid=grid,
        in_specs=[
            pl.BlockSpec((blk, EMB), lambda i: (i, 0)),
            full((EMB, 3 * EMB)),
            full((1, 3 * EMB)),
            full((2 * HD, HD)),
            full((1, HD)),
        ],
        out_specs=[
            pl.BlockSpec((NC, blk, EMB), lambda i: (0, i, 0)),
            pl.BlockSpec((blk, EMB), lambda i: (i, 0)),
        ],
        out_shape=[
            jax.ShapeDtypeStruct((NC, N_PAD, EMB), jnp.float32),
            jax.ShapeDtypeStruct((N_PAD, EMB), jnp.float32),
        ],
    )(x_pad, w_kqv, b_kqv, w_att1, b_att1)


def _edge_kernel(s_hbm, aq_hbm, src2_hbm, dst2_hbm, w2_hbm, z_hbm,
                 acc_hbm,
                 src_v, dst_v, s_rows, aq_rows, w2_v,
                 acc_sh, sem_s, sem_q, sem_i):
    cc = lax.axis_index("c")
    sid = lax.axis_index("s")
    row_base = sid * K

    pltpu.sync_copy(z_hbm, acc_sh.at[pl.ds(sid * ROWS_PER_SUB, ROWS_PER_SUB)])
    pltpu.sync_copy(w2_hbm, w2_v)
    plsc.subcore_barrier()

    w2v = [w2_v[pl.ds(16 * j, 16)] for j in range(4)]
    lane = lax.iota(jnp.int32, 16)
    perms = [jnp.bitwise_xor(lane, sh) for sh in (8, 4, 2, 1)]
    tail_base = jnp.where(lane == 1, 1.0, 0.0).astype(jnp.float32)
    aq_off = cc * HD

    def hsum(u):
        for p in perms:
            u = u + u.at[p].get(mode="promise_in_bounds")
        return u

    def fetch_idx(k, s):
        pltpu.async_copy(src2_hbm.at[row_base + k], src_v.at[s], sem_i.at[s])
        pltpu.async_copy(dst2_hbm.at[row_base + k], dst_v.at[s], sem_i.at[s])

    def wait_idx(k, s):
        pltpu.make_async_copy(src2_hbm.at[row_base + k],
                              src_v.at[s], sem_i.at[s]).wait()
        pltpu.make_async_copy(dst2_hbm.at[row_base + k],
                              dst_v.at[s], sem_i.at[s]).wait()

    def gather(s, b):
        pltpu.async_copy(s_hbm.at[cc].at[src_v.at[s]], s_rows.at[b],
                         sem_s.at[b])
        pltpu.async_copy(aq_hbm.at[dst_v.at[s]], aq_rows.at[b], sem_q.at[b])

    def wait_gather(s, b):
        pltpu.make_async_copy(s_hbm.at[cc].at[src_v.at[s]],
                              s_rows.at[b], sem_s.at[b]).wait()
        pltpu.make_async_copy(aq_hbm.at[dst_v.at[s]],
                              aq_rows.at[b], sem_q.at[b]).wait()

    def compute(b):
        @plsc.parallel_loop(0, C, unroll=4)
        def edge_body(e):
            t = [jnp.maximum(s_rows[b, e, pl.ds(16 * j, 16)]
                             + aq_rows[b, e, pl.ds(aq_off + 16 * j, 16)], 0.0)
                 * w2v[j] for j in range(4)]
            wv = jnp.exp(hsum((t[0] + t[1]) + (t[2] + t[3])))
            vv = [s_rows[b, e, pl.ds(64 + 16 * j, 16)] for j in range(4)]
            for j in range(4):
                s_rows[b, e, pl.ds(16 * j, 16)] = vv[j] * wv
            s_rows[b, e, pl.ds(64, 16)] = jnp.where(lane == 0, wv, tail_base)

    for s in range(4):
        fetch_idx(s, s)
    wait_idx(0, 0)
    gather(0, 0)
    wait_idx(1, 1)
    gather(1, 1)

    def quad_body(q, carry):
        k0 = q * 4
        for i in range(4):
            k = k0 + i
            b = i % 2
            wait_gather(i, b)
            compute(b)
            pltpu.sync_copy(s_rows.at[b], acc_sh.at[dst_v.at[i]], add=True)

            @pl.when(k + 2 < K)
            def _():
                wait_idx(k + 2, (i + 2) % 4)
                gather((i + 2) % 4, b)

            @pl.when(k + 4 < K)
            def _():
                fetch_idx(k + 4, i)
        return carry

    lax.fori_loop(0, K // 4, quad_body, 0)
    plsc.subcore_barrier()
    pltpu.sync_copy(acc_sh.at[pl.ds(sid * ROWS_PER_SUB, ROWS_PER_SUB)],
                    acc_hbm.at[cc, pl.ds(sid * ROWS_PER_SUB, ROWS_PER_SUB)])


def _edge_pass(s_tab, aq_tab, src2, dst2, w2t, zrows):
    mesh = plsc.VectorSubcoreMesh(core_axis_name="c", subcore_axis_name="s")
    f = pl.kernel(
        _edge_kernel,
        out_type=jax.ShapeDtypeStruct((NC, N_PAD, EMB), jnp.float32),
        mesh=mesh,
        scratch_types=[
            pltpu.VMEM((4, C), jnp.int32),
            pltpu.VMEM((4, C), jnp.int32),
            pltpu.VMEM((2, C, EMB), jnp.float32),
            pltpu.VMEM((2, C, EMB), jnp.float32),
            pltpu.VMEM((HD,), jnp.float32),
            pltpu.VMEM_SHARED((N_ACC, EMB), jnp.float32),
            pltpu.SemaphoreType.DMA((2,)),
            pltpu.SemaphoreType.DMA((2,)),
            pltpu.SemaphoreType.DMA((4,)),
        ],
    )
    return f(s_tab, aq_tab, src2, dst2, w2t, zrows)


def _out_kernel(acc_ref, x_ref, wout_ref, bout_ref, o_ref):
    a0 = acc_ref[0]
    a1 = acc_ref[1]
    norm = jnp.concatenate([a0[:, 0:64] / (a0[:, 64:65] + 1e-16),
                            a1[:, 0:64] / (a1[:, 64:65] + 1e-16)], axis=1)
    cnt = a0[:, 65:66]
    o = (jnp.dot(norm, wout_ref[...], preferred_element_type=jnp.float32)
         + cnt * bout_ref[...])
    o_ref[...] = jnp.maximum(o, 0.0) + x_ref[...]


def _finish(acc, x_pad, w_out, b_out):
    blk = 512
    grid = (N_PAD // blk,)
    return pl.pallas_call(
        _out_kernel,
        grid=grid,
        in_specs=[
            pl.BlockSpec((NC, blk, EMB), lambda i: (0, i, 0)),
            pl.BlockSpec((blk, EMB), lambda i: (i, 0)),
            pl.BlockSpec((EMB, EMB), lambda i: (0, 0)),
            pl.BlockSpec((1, EMB), lambda i: (0, 0)),
        ],
        out_specs=pl.BlockSpec((blk, EMB), lambda i: (i, 0)),
        out_shape=jax.ShapeDtypeStruct((N_PAD, EMB), jnp.float32),
    )(acc, x_pad, w_out, b_out)


def kernel(x, edge_index, W_kqv, b_kqv, W_att1, b_att1, W_att2, b_att2,
           W_out, b_out):
    n = x.shape[0]
    ei = jnp.concatenate([edge_index, edge_index[::-1]], axis=1)
    mask = ei[0] != ei[1]
    src = jnp.where(mask, ei[0], jnp.zeros((), jnp.int32))
    dst = jnp.where(mask, ei[1], jnp.asarray(n, jnp.int32))
    loops = jnp.arange(n, dtype=jnp.int32)
    src = jnp.concatenate([src, loops])
    dst = jnp.concatenate([dst, loops])
    pad = E_PAD - src.shape[0]
    src = jnp.concatenate([src, jnp.zeros((pad,), jnp.int32)]).reshape(-1, C)
    dst = jnp.concatenate([dst, jnp.full((pad,), n + 1, jnp.int32)]).reshape(-1, C)

    x_pad = jnp.pad(x, ((0, N_PAD - n), (0, 0)))
    w2t = W_att2[:, 0]
    zrows = jnp.zeros((ROWS_PER_SUB, EMB), jnp.float32)

    s_tab, aq_tab = _project(x_pad, W_kqv, b_kqv.reshape(1, -1),
                             W_att1, b_att1.reshape(1, -1))
    acc = _edge_pass(s_tab, aq_tab, src, dst, w2t, zrows)
    out = _finish(acc, x_pad, W_out, b_out.reshape(1, -1))
    return out[:n]
